# Initial kernel scaffold; baseline (speedup 1.0000x reference)
import jax
import jax.numpy as jnp
from jax.experimental import pallas as pl


def kernel(rank_embed, file_embed, batch_size):
    raise NotImplementedError("write your pallas kernel here")


# speedup vs baseline: 46.8810x; 46.8810x over previous
import jax
import jax.numpy as jnp
from jax.experimental import pallas as pl

_B = 16384
_P = 64
_D = 128
_BLK = 256


def _body(rank_ref, file_ref, out_ref):
    r = rank_ref[...]
    f = file_ref[...]
    rank_part = jnp.broadcast_to(r[:, None, :], (8, 8, _D))
    file_part = jnp.broadcast_to(f[None, :, :], (8, 8, _D))
    table = (rank_part + file_part).reshape(_P, _D)
    out_ref[...] = jnp.broadcast_to(table[None, :, :], (_BLK, _P, _D))


def kernel(rank_embed, file_embed, batch_size):
    return pl.pallas_call(
        _body,
        grid=(_B // _BLK,),
        in_specs=[
            pl.BlockSpec((8, _D), lambda i: (0, 0)),
            pl.BlockSpec((8, _D), lambda i: (0, 0)),
        ],
        out_specs=pl.BlockSpec((_BLK, _P, _D), lambda i: (i, 0, 0)),
        out_shape=jax.ShapeDtypeStruct((_B, _P, _D), jnp.float32),
    )(rank_embed, file_embed)
